# baseline (speedup 1.0000x reference)
import functools

import jax
import jax.numpy as jnp
from jax import lax
from jax.experimental import pallas as pl
from jax.experimental.pallas import tpu as pltpu
from jax.experimental.pallas import tpu_sc as plsc

EMBED_DIM = 64
LANES = 16
N_WORKERS = 32
CHUNK = 128
ENTITY_AWARE_COFF = 0.001


def _sc_body(x_hbm, idx_hbm, pd_hbm, id_hbm, ud_hbm,
             idx_v, rows_v, pd_v, id_v, ud_v, z_v, sem):
    per_w = pd_v.shape[0]
    batch = pd_hbm.shape[0]
    n_chunks = per_w // CHUNK
    wid = lax.axis_index("s") * 2 + lax.axis_index("c")
    base_w = wid * per_w

    for k in range(7):
        pltpu.sync_copy(idx_hbm.at[pl.ds(k * batch + base_w, per_w)],
                        idx_v.at[pl.ds(k * per_w, per_w)])

    for c in range(n_chunks):
        cbase = c * CHUNK
        cps = [
            pltpu.async_copy(
                x_hbm.at[idx_v.at[pl.ds(k * per_w + cbase, CHUNK)]],
                rows_v.at[k], sem)
            for k in range(7)
        ]
        for cp in cps:
            cp.wait()

        def elem(e, carry):
            pd = None
            idp = None
            udp = None
            for j in range(EMBED_DIM // LANES):
                sl = pl.ds(j * LANES, LANES)
                uu = rows_v[0, e, sl]
                pp = rows_v[1, e, sl]
                nn = rows_v[2, e, sl]
                pe = rows_v[3, e, sl]
                ne = rows_v[4, e, sl]
                pu = rows_v[5, e, sl]
                nu = rows_v[6, e, sl]
                t_pd = uu * (pp - nn)
                a = pp - pe
                b = pp - ne
                t_id = a * a - b * b
                a2 = uu - pu
                b2 = uu - nu
                t_ud = a2 * a2 - b2 * b2
                pd = t_pd if pd is None else pd + t_pd
                idp = t_id if idp is None else idp + t_id
                udp = t_ud if udp is None else udp + t_ud
            pd_v[cbase + e, :] = pd
            id_v[cbase + e, :] = idp
            ud_v[cbase + e, :] = udp
            return carry

        lax.fori_loop(0, CHUNK, elem, 0)

    def group(g, carry):
        ids = g * LANES + lax.iota(jnp.int32, LANES)
        for t, part in enumerate((pd_v, id_v, ud_v)):
            z = None
            for l in range(LANES):
                col = plsc.load_gather(
                    part, [ids, jnp.full((LANES,), l, jnp.int32)])
                z = col if z is None else z + col
            z_v[t, pl.ds(g * LANES, LANES)] = z
        return carry

    lax.fori_loop(0, per_w // LANES, group, 0)

    pltpu.sync_copy(z_v.at[0], pd_hbm.at[pl.ds(base_w, per_w)])
    pltpu.sync_copy(z_v.at[1], id_hbm.at[pl.ds(base_w, per_w)])
    pltpu.sync_copy(z_v.at[2], ud_hbm.at[pl.ds(base_w, per_w)])


def _sc_partials(x, idx7):
    batch = idx7.shape[0] // 7
    per_w = batch // N_WORKERS
    mesh = plsc.VectorSubcoreMesh(core_axis_name="c", subcore_axis_name="s")
    out = jax.ShapeDtypeStruct((batch,), jnp.float32)
    f = functools.partial(
        pl.kernel,
        out_type=[out, out, out],
        mesh=mesh,
        scratch_types=[
            pltpu.VMEM((7 * per_w,), jnp.int32),
            pltpu.VMEM((7, CHUNK, EMBED_DIM), jnp.float32),
            pltpu.VMEM((per_w, LANES), jnp.float32),
            pltpu.VMEM((per_w, LANES), jnp.float32),
            pltpu.VMEM((per_w, LANES), jnp.float32),
            pltpu.VMEM((3, per_w), jnp.float32),
            pltpu.SemaphoreType.DMA,
        ],
        compiler_params=pltpu.CompilerParams(
            use_tc_tiling_on_sc=False, needs_layout_passes=False),
    )(_sc_body)
    return f(x, idx7)


def _tc_body(z_ref, zi_ref, zu_ref, mi_ref, mu_ref, out_ref):
    z = z_ref[...]
    zi = zi_ref[...] * mi_ref[...]
    zu = zu_ref[...] * mu_ref[...]

    def nls(t):
        mt = jnp.maximum(-t, 0.0)
        return mt + jnp.log(jnp.exp(-t - mt) + jnp.exp(-mt))

    cf = jnp.sum(nls(z))
    reg = jnp.sum(nls(zi)) + jnp.sum(nls(zu))
    out_ref[0, 0] = cf + ENTITY_AWARE_COFF * reg


def _tc_finish(z, zi, zu, mi, mu):
    batch = z.shape[0]
    rows = 128
    cols = batch // rows
    out = pl.pallas_call(
        _tc_body,
        out_shape=jax.ShapeDtypeStruct((1, 1), jnp.float32),
        out_specs=pl.BlockSpec(memory_space=pltpu.SMEM),
    )(z.reshape(rows, cols), zi.reshape(rows, cols), zu.reshape(rows, cols),
      mi.reshape(rows, cols), mu.reshape(rows, cols))
    return out[0, 0]


def kernel(x, pos_neg_pair_t):
    p = pos_neg_pair_t.astype(jnp.int32)
    cols = p.T
    idx7 = jnp.concatenate(
        [cols[0:5], cols[6:8]], axis=0).reshape(-1)
    mi = cols[5].astype(jnp.float32)
    mu = cols[8].astype(jnp.float32)
    z, zi, zu = _sc_partials(x, idx7)
    return _tc_finish(z, zi, zu, mi, mu)
